# baseline (speedup 1.0000x reference)
import functools

import jax
import jax.numpy as jnp
from jax import lax
from jax.experimental import pallas as pl
from jax.experimental.pallas import tpu as pltpu
from jax.experimental.pallas import tpu_sc as plsc

D = 1024
NT = 4 * 2048
NW = 32
TPW = NT // NW
T = 8
R = 6 * T
NCHUNK = TPW // T
LANES = 16


def _spatial_body(c_hbm, w_hbm, out_hbm,
                  idx_v, ra, rb, oa, ob, ga, gb, soa, sob):
    rows = (ra, rb)
    outs = (oa, ob)
    gsems = (ga, gb)
    osems = (soa, sob)
    wid = lax.axis_index("s") * 2 + lax.axis_index("c")
    base = wid * TPW

    pltpu.sync_copy(c_hbm.at[wid], idx_v)

    def gather_start(c, s):
        pltpu.async_copy(w_hbm.at[idx_v.at[pl.ds(c * R, R)]],
                         rows[s], gsems[s])

    def gather_wait(s):
        pltpu.make_async_copy(w_hbm.at[idx_v.at[pl.ds(0, R)]],
                              rows[s], gsems[s]).wait()

    def combine_store(c, s):
        r = rows[s]
        o = outs[s]

        def tok_body(t, carry):
            def elem_body(e, carry2):
                sl = pl.ds(e * LANES, LANES)
                o[t, sl] = ((r[t, sl] + r[T + t, sl])
                            + (r[2 * T + t, sl] + r[3 * T + t, sl])
                            + (r[4 * T + t, sl] + r[5 * T + t, sl]))
                return carry2
            return lax.fori_loop(0, D // LANES, elem_body, carry, unroll=8)

        lax.fori_loop(0, T, tok_body, 0)
        pltpu.async_copy(o, out_hbm.at[pl.ds(base + c * T, T)], osems[s])

    def out_wait(s):
        pltpu.make_async_copy(outs[s], out_hbm.at[pl.ds(base, T)],
                              osems[s]).wait()

    gather_start(0, 0)
    gather_start(1, 1)
    gather_wait(0)
    combine_store(0, 0)
    gather_start(2, 0)
    gather_wait(1)
    combine_store(1, 1)
    gather_start(3, 1)

    def pair_body(k, carry):
        c0 = k * 2
        gather_wait(0)
        out_wait(0)
        combine_store(c0, 0)
        gather_start(c0 + 2, 0)
        gather_wait(1)
        out_wait(1)
        combine_store(c0 + 1, 1)
        gather_start(c0 + 3, 1)
        return carry

    lax.fori_loop(1, NCHUNK // 2 - 1, pair_body, 0)

    gather_wait(0)
    out_wait(0)
    combine_store(NCHUNK - 2, 0)
    gather_wait(1)
    out_wait(1)
    combine_store(NCHUNK - 1, 1)
    out_wait(0)
    out_wait(1)


_spatial = functools.partial(
    pl.kernel,
    mesh=plsc.VectorSubcoreMesh(core_axis_name="c", subcore_axis_name="s"),
    out_type=jax.ShapeDtypeStruct((NT, D), jnp.float32),
    scratch_types=[pltpu.VMEM((NCHUNK * R,), jnp.int32),
                   pltpu.VMEM((R, D), jnp.float32),
                   pltpu.VMEM((R, D), jnp.float32),
                   pltpu.VMEM((T, D), jnp.float32),
                   pltpu.VMEM((T, D), jnp.float32)]
                  + [pltpu.SemaphoreType.DMA for _ in range(4)],
)(_spatial_body)


def kernel(coordinates, W_tlx, W_tly, W_brx, W_bry, W_w, W_h):
    b, s, _ = coordinates.shape
    w_all = jnp.concatenate((W_tlx, W_tly, W_brx, W_bry, W_w, W_h), axis=0)
    coords = coordinates.astype(jnp.int32).reshape(NT, 6)
    coords = coords + jnp.arange(6, dtype=jnp.int32) * 1024
    idx = coords.reshape(NW, NCHUNK, T, 6).transpose(0, 1, 3, 2)
    idx = idx.reshape(NW, NCHUNK * R)
    out = _spatial(idx, w_all)
    return out.reshape(b, s, D)
